# Initial kernel scaffold; baseline (speedup 1.0000x reference)
import jax
import jax.numpy as jnp
from jax.experimental import pallas as pl


def kernel(input_ids, W):
    raise NotImplementedError("write your pallas kernel here")


# speedup vs baseline: 175.6792x; 175.6792x over previous
import jax
import jax.numpy as jnp
from jax import lax
from jax.experimental import pallas as pl
from jax.experimental.pallas import tpu as pltpu

_NUM_BUCKETS = 32
_MAX_DISTANCE = 128
_NUM_HEADS = 16
_S = 2048
_TI = 256
_L = 4224


def _bucket_map(d):
    nb = _NUM_BUCKETS // 2
    ret = jnp.where(d > 0, nb, 0).astype(jnp.int32)
    rp = jnp.abs(d)
    max_exact = nb // 2
    is_small = rp < max_exact
    rel_f = rp.astype(jnp.float32)
    scale = (nb - max_exact) / jnp.log(_MAX_DISTANCE / max_exact)
    large = max_exact + (
        jnp.log(jnp.maximum(rel_f, 1.0) / max_exact) * scale
    ).astype(jnp.int32)
    large = jnp.minimum(large, nb - 1)
    return ret + jnp.where(is_small, rp, large)


def _kernel_body(w_ref, out_ref, bucket8_ref, v8_ref):
    h = pl.program_id(0)

    @pl.when(h == 0)
    def _init_buckets():
        m = lax.broadcasted_iota(jnp.int32, (8, _L), 1)
        r = lax.broadcasted_iota(jnp.int32, (8, _L), 0)
        bucket8_ref[...] = _bucket_map(m - r - (_S - 1))

    bucket8 = bucket8_ref[...]
    acc = jnp.zeros((8, _L), jnp.float32)
    for b in range(_NUM_BUCKETS):
        acc = jnp.where(bucket8 == b, w_ref[b, h], acc)
    v8_ref[...] = acc

    for s in range(_S // 8):
        c0 = (_S - 1) - 8 * s
        out_ref[0, 0, 8 * s:8 * s + 8, :] = v8_ref[:, c0:c0 + _S]


def kernel(input_ids, W):
    S = input_ids.shape[1]
    assert S == _S and W.shape == (_NUM_BUCKETS, _NUM_HEADS)
    out = pl.pallas_call(
        _kernel_body,
        grid=(_NUM_HEADS,),
        in_specs=[pl.BlockSpec(memory_space=pltpu.SMEM)],
        out_specs=pl.BlockSpec(
            (1, 1, _S, _S), lambda h: (0, h, 0, 0)
        ),
        out_shape=jax.ShapeDtypeStruct((1, _NUM_HEADS, _S, _S), jnp.float32),
        scratch_shapes=[
            pltpu.VMEM((8, _L), jnp.int32),
            pltpu.VMEM((8, _L), jnp.float32),
        ],
        compiler_params=pltpu.CompilerParams(
            dimension_semantics=("arbitrary",),
        ),
    )(W.astype(jnp.float32))
    return out


# speedup vs baseline: 187.9663x; 1.0699x over previous
import jax
import jax.numpy as jnp
from jax import lax
from jax.experimental import pallas as pl
from jax.experimental.pallas import tpu as pltpu

_NUM_BUCKETS = 32
_MAX_DISTANCE = 128
_NUM_HEADS = 16
_S = 2048
_TI = 256
_L = 4224


def _bucket_map(d):
    nb = _NUM_BUCKETS // 2
    ret = jnp.where(d > 0, nb, 0).astype(jnp.int32)
    rp = jnp.abs(d)
    max_exact = nb // 2
    is_small = rp < max_exact
    rel_f = rp.astype(jnp.float32)
    scale = (nb - max_exact) / jnp.log(_MAX_DISTANCE / max_exact)
    large = max_exact + (
        jnp.log(jnp.maximum(rel_f, 1.0) / max_exact) * scale
    ).astype(jnp.int32)
    large = jnp.minimum(large, nb - 1)
    return ret + jnp.where(is_small, rp, large)


def _kernel_body(w_ref, out_ref, bucket8_ref, v8r_ref):
    h = pl.program_id(0)

    @pl.when(h == 0)
    def _init_buckets():
        m = lax.broadcasted_iota(jnp.int32, (8, _L), 1)
        r = lax.broadcasted_iota(jnp.int32, (8, _L), 0)
        bucket8_ref[...] = _bucket_map(m - r - (_S - 1))

    bucket8 = bucket8_ref[...]
    acc = jnp.zeros((8, _L), jnp.float32)
    for b in range(_NUM_BUCKETS):
        acc = jnp.where(bucket8 == b, w_ref[b, h], acc)
    for k in range(16):
        rho = 8 * k + 7
        v8r_ref[k] = jnp.concatenate([acc[:, rho:], acc[:, :rho]], axis=1)

    for s in range(_S // 8):
        c0 = (_S - 1) - 8 * s
        rho = c0 % 128
        a = c0 - rho
        out_ref[0, 0, 8 * s:8 * s + 8, :] = v8r_ref[(rho - 7) // 8, :, a:a + _S]


def kernel(input_ids, W):
    S = input_ids.shape[1]
    assert S == _S and W.shape == (_NUM_BUCKETS, _NUM_HEADS)
    out = pl.pallas_call(
        _kernel_body,
        grid=(_NUM_HEADS,),
        in_specs=[pl.BlockSpec(memory_space=pltpu.SMEM)],
        out_specs=pl.BlockSpec(
            (1, 1, _S, _S), lambda h: (0, h, 0, 0)
        ),
        out_shape=jax.ShapeDtypeStruct((1, _NUM_HEADS, _S, _S), jnp.float32),
        scratch_shapes=[
            pltpu.VMEM((8, _L), jnp.int32),
            pltpu.VMEM((16, 8, _L), jnp.float32),
        ],
        compiler_params=pltpu.CompilerParams(
            dimension_semantics=("arbitrary",),
        ),
    )(W.astype(jnp.float32))
    return out
